# baseline (speedup 1.0000x reference)
import jax
import jax.numpy as jnp
from jax import lax
from jax.experimental import pallas as pl
from jax.experimental.pallas import tpu as pltpu
from jax.experimental.pallas import tpu_sc as plsc

N = 10000
E = 320000
D = 128
H = 128
G = 64
C = 2

NC = 2
NS = 16
NW = NC * NS
NP = 10240
FS = H // NW
EDEG = E // NW
EW = 4096
NWINE = 79
EPADA = NWINE * EW

_mesh = plsc.VectorSubcoreMesh(
    core_axis_name="c", subcore_axis_name="s", num_cores=NC, num_subcores=NS)


def _deg_body(dst_hbm, zeros_hbm, out_hbm, dst_v, deg_v):
    c = lax.axis_index("c")
    s = lax.axis_index("s")
    wid = s * NC + c
    pltpu.sync_copy(dst_hbm.at[wid], dst_v)
    pltpu.sync_copy(zeros_hbm, deg_v)
    ones = jnp.ones((16,), jnp.float32)

    def body(k, carry):
        idx = dst_v[pl.ds(k * 16, 16)]
        plsc.addupdate_scatter(deg_v, [idx], ones)
        return carry

    lax.fori_loop(0, EDEG // 16, body, 0)
    pltpu.sync_copy(deg_v, out_hbm.at[wid])


_deg = pl.kernel(
    _deg_body,
    out_type=jax.ShapeDtypeStruct((NW, NP), jnp.float32),
    mesh=_mesh,
    compiler_params=pltpu.CompilerParams(needs_layout_passes=False),
    scratch_types=[
        pltpu.VMEM((EDEG,), jnp.int32),
        pltpu.VMEM((NP,), jnp.float32),
    ],
)


def _agg_body(yt_hbm, eidx_hbm, zeros_hbm, out_hbm,
              ytab, acc, ib0, ib1, sem0, sem1):
    c = lax.axis_index("c")
    s = lax.axis_index("s")
    wid = s * NC + c
    pltpu.async_copy(eidx_hbm.at[0], ib0, sem0)
    pltpu.sync_copy(yt_hbm.at[wid], ytab)
    pltpu.sync_copy(zeros_hbm, acc)

    frows = [jnp.full((16,), f, jnp.int32) for f in range(FS)]

    def process(ib):
        def grp(g, carry):
            src16 = ib[0, pl.ds(g * 16, 16)]
            dst16 = ib[1, pl.ds(g * 16, 16)]
            for f in range(FS):
                v = plsc.load_gather(ytab, [frows[f], src16])
                plsc.addupdate_scatter(acc, [frows[f], dst16], v)
            return carry

        lax.fori_loop(0, EW // 16, grp, 0)

    def dbl(k, carry):
        w0 = 2 * k
        pltpu.make_async_copy(eidx_hbm.at[w0], ib0, sem0).wait()
        pltpu.async_copy(eidx_hbm.at[w0 + 1], ib1, sem1)
        process(ib0)
        pltpu.make_async_copy(eidx_hbm.at[w0 + 1], ib1, sem1).wait()
        pltpu.async_copy(eidx_hbm.at[w0 + 2], ib0, sem0)
        process(ib1)
        return carry

    lax.fori_loop(0, NWINE // 2, dbl, 0)
    pltpu.make_async_copy(eidx_hbm.at[NWINE - 1], ib0, sem0).wait()
    process(ib0)
    pltpu.sync_copy(acc, out_hbm.at[wid])


_agg = pl.kernel(
    _agg_body,
    out_type=jax.ShapeDtypeStruct((NW, FS, NP), jnp.float32),
    mesh=_mesh,
    compiler_params=pltpu.CompilerParams(needs_layout_passes=False),
    scratch_types=[
        pltpu.VMEM((FS, NP), jnp.float32),
        pltpu.VMEM((FS, NP), jnp.float32),
        pltpu.VMEM((2, EW), jnp.int32),
        pltpu.VMEM((2, EW), jnp.int32),
        pltpu.SemaphoreType.DMA,
        pltpu.SemaphoreType.DMA,
    ],
)


_R = 2048


def _tc1_body(degp_ref, x_ref, w_ref, dinv_ref, yt_ref):
    deg = jnp.sum(degp_ref[...], axis=0, keepdims=True) + 1.0
    dinv = lax.rsqrt(deg)
    dinv_ref[...] = dinv
    yt = lax.dot_general(w_ref[...], x_ref[...], (((0,), (1,)), ((), ())),
                         preferred_element_type=jnp.float32)
    yt_ref[...] = yt * dinv


def _tc1(degp, xp, w):
    return pl.pallas_call(
        _tc1_body,
        grid=(NP // _R,),
        in_specs=[
            pl.BlockSpec((NW, _R), lambda i: (0, i)),
            pl.BlockSpec((_R, D), lambda i: (i, 0)),
            pl.BlockSpec((D, H), lambda i: (0, 0)),
        ],
        out_specs=[
            pl.BlockSpec((1, _R), lambda i: (0, i)),
            pl.BlockSpec((H, _R), lambda i: (0, i)),
        ],
        out_shape=[
            jax.ShapeDtypeStruct((1, NP), jnp.float32),
            jax.ShapeDtypeStruct((H, NP), jnp.float32),
        ],
    )(degp, xp, w)


def _tcmid_body(st_ref, yt_ref, dinv_ref, b_ref, w_ref, ynt_ref):
    dinv = dinv_ref[...]
    z = st_ref[...] + yt_ref[...]
    h = jnp.maximum(z * dinv + b_ref[...], 0.0)
    yn = lax.dot_general(w_ref[...], h, (((0,), (0,)), ((), ())),
                         preferred_element_type=jnp.float32)
    ynt_ref[...] = yn * dinv


def _tcmid(st, yt, dinv, b, w):
    return pl.pallas_call(
        _tcmid_body,
        grid=(NP // _R,),
        in_specs=[
            pl.BlockSpec((H, _R), lambda i: (0, i)),
            pl.BlockSpec((H, _R), lambda i: (0, i)),
            pl.BlockSpec((1, _R), lambda i: (0, i)),
            pl.BlockSpec((H, 1), lambda i: (0, 0)),
            pl.BlockSpec((H, H), lambda i: (0, 0)),
        ],
        out_specs=pl.BlockSpec((H, _R), lambda i: (0, i)),
        out_shape=jax.ShapeDtypeStruct((H, NP), jnp.float32),
    )(st, yt, dinv, b, w)


def _tcf_body(st_ref, yt_ref, dinv_ref, b_ref, batch_ref, wlin_ref, blin_ref,
              out_ref, pooled_acc, cnt_acc):
    i = pl.program_id(0)

    @pl.when(i == 0)
    def _():
        pooled_acc[...] = jnp.zeros_like(pooled_acc)
        cnt_acc[...] = jnp.zeros_like(cnt_acc)

    z = (st_ref[...] + yt_ref[...]) * dinv_ref[...] + b_ref[...]
    bb = batch_ref[...]
    gi = lax.broadcasted_iota(jnp.int32, (1, G), 1)
    mt = (bb == gi).astype(jnp.float32)
    pooled_acc[...] += jnp.dot(z, mt, preferred_element_type=jnp.float32)
    cnt_acc[...] += jnp.sum(mt, axis=0, keepdims=True)

    @pl.when(i == pl.num_programs(0) - 1)
    def _():
        pooled = pooled_acc[...] / jnp.maximum(cnt_acc[...], 1.0)
        out_ref[...] = lax.dot_general(
            pooled, wlin_ref[...], (((0,), (0,)), ((), ())),
            preferred_element_type=jnp.float32) + blin_ref[...]


def _tcf(st, yt, dinv, b, batchp, wlin, blin):
    return pl.pallas_call(
        _tcf_body,
        grid=(NP // _R,),
        in_specs=[
            pl.BlockSpec((H, _R), lambda i: (0, i)),
            pl.BlockSpec((H, _R), lambda i: (0, i)),
            pl.BlockSpec((1, _R), lambda i: (0, i)),
            pl.BlockSpec((H, 1), lambda i: (0, 0)),
            pl.BlockSpec((_R, 1), lambda i: (i, 0)),
            pl.BlockSpec((H, C), lambda i: (0, 0)),
            pl.BlockSpec((1, C), lambda i: (0, 0)),
        ],
        out_specs=pl.BlockSpec((G, C), lambda i: (0, 0)),
        out_shape=jax.ShapeDtypeStruct((G, C), jnp.float32),
        scratch_shapes=[
            pltpu.VMEM((H, G), jnp.float32),
            pltpu.VMEM((1, G), jnp.float32),
        ],
    )(st, yt, dinv, b, batchp, wlin, blin)


def kernel(x, edge_index, batch, W1, b1, W2, b2, W3, b3, Wlin, blin):
    dstd_r = edge_index[1].reshape(NW, EDEG)
    eidx_r = jnp.pad(edge_index, ((0, 0), (0, EPADA - E)),
                     constant_values=N).reshape(2, NWINE, EW).transpose(1, 0, 2)
    xp = jnp.pad(x, ((0, NP - N), (0, 0)))
    batchp = jnp.pad(batch, (0, NP - N), constant_values=G).reshape(NP, 1)
    zeros1d = jnp.zeros((NP,), jnp.float32)
    zerosfs = jnp.zeros((FS, NP), jnp.float32)

    degp = _deg(dstd_r, zeros1d)
    dinv, yt = _tc1(degp, xp, W1)
    st = _agg(yt.reshape(NW, FS, NP), eidx_r, zerosfs).reshape(H, NP)
    yt = _tcmid(st, yt, dinv, b1.reshape(H, 1), W2)
    st = _agg(yt.reshape(NW, FS, NP), eidx_r, zerosfs).reshape(H, NP)
    yt = _tcmid(st, yt, dinv, b2.reshape(H, 1), W3)
    st = _agg(yt.reshape(NW, FS, NP), eidx_r, zerosfs).reshape(H, NP)
    return _tcf(st, yt, dinv, b3.reshape(H, 1), batchp,
                Wlin, blin.reshape(1, C))


# speedup vs baseline: 2.4298x; 2.4298x over previous
import jax
import jax.numpy as jnp
from jax import lax
from jax.experimental import pallas as pl
from jax.experimental.pallas import tpu as pltpu
from jax.experimental.pallas import tpu_sc as plsc

N = 10000
E = 320000
D = 128
H = 128
G = 64
C = 2

NC = 2
NS = 16
NW = NC * NS
NP = 10240
FS = H // NW
EDEG = E // NW
EW = 4096
NWINE = 79
EPADA = NWINE * EW

_mesh = plsc.VectorSubcoreMesh(
    core_axis_name="c", subcore_axis_name="s", num_cores=NC, num_subcores=NS)


def _deg_body(dst_hbm, zeros_hbm, out_hbm, dst_v, deg_v):
    c = lax.axis_index("c")
    s = lax.axis_index("s")
    wid = s * NC + c
    pltpu.sync_copy(dst_hbm.at[wid], dst_v)
    pltpu.sync_copy(zeros_hbm, deg_v)
    ones = jnp.ones((16,), jnp.float32)

    def body(k, carry):
        idx = dst_v[pl.ds(k * 16, 16)]
        plsc.addupdate_scatter(deg_v, [idx], ones)
        return carry

    lax.fori_loop(0, EDEG // 16, body, 0)
    pltpu.sync_copy(deg_v, out_hbm.at[wid])


_deg = pl.kernel(
    _deg_body,
    out_type=jax.ShapeDtypeStruct((NW, NP), jnp.float32),
    mesh=_mesh,
    compiler_params=pltpu.CompilerParams(needs_layout_passes=False),
    scratch_types=[
        pltpu.VMEM((EDEG,), jnp.int32),
        pltpu.VMEM((NP,), jnp.float32),
    ],
)


def _agg_body(yt_hbm, eidx_hbm, zeros_hbm, out_hbm,
              ytab, acc, ib0, ib1, sem0, sem1):
    c = lax.axis_index("c")
    s = lax.axis_index("s")
    wid = s * NC + c
    pltpu.async_copy(eidx_hbm.at[0], ib0, sem0)
    pltpu.sync_copy(yt_hbm.at[wid], ytab)
    pltpu.sync_copy(zeros_hbm, acc)

    frows = [jnp.full((16,), f, jnp.int32) for f in range(FS)]

    def process(ib):
        @plsc.parallel_loop(0, EW, step=16, unroll=4)
        def grp(i):
            src16 = ib[0, pl.ds(i, 16)]
            dst16 = ib[1, pl.ds(i, 16)]
            for f in range(FS):
                v = plsc.load_gather(ytab, [frows[f], src16])
                plsc.addupdate_scatter(acc, [frows[f], dst16], v)

    def dbl(k, carry):
        w0 = 2 * k
        pltpu.make_async_copy(eidx_hbm.at[w0], ib0, sem0).wait()
        pltpu.async_copy(eidx_hbm.at[w0 + 1], ib1, sem1)
        process(ib0)
        pltpu.make_async_copy(eidx_hbm.at[w0 + 1], ib1, sem1).wait()
        pltpu.async_copy(eidx_hbm.at[w0 + 2], ib0, sem0)
        process(ib1)
        return carry

    lax.fori_loop(0, NWINE // 2, dbl, 0)
    pltpu.make_async_copy(eidx_hbm.at[NWINE - 1], ib0, sem0).wait()
    process(ib0)
    pltpu.sync_copy(acc, out_hbm.at[wid])


_agg = pl.kernel(
    _agg_body,
    out_type=jax.ShapeDtypeStruct((NW, FS, NP), jnp.float32),
    mesh=_mesh,
    compiler_params=pltpu.CompilerParams(needs_layout_passes=False),
    scratch_types=[
        pltpu.VMEM((FS, NP), jnp.float32),
        pltpu.VMEM((FS, NP), jnp.float32),
        pltpu.VMEM((2, EW), jnp.int32),
        pltpu.VMEM((2, EW), jnp.int32),
        pltpu.SemaphoreType.DMA,
        pltpu.SemaphoreType.DMA,
    ],
)


_R = 2048


def _tc1_body(degp_ref, x_ref, w_ref, dinv_ref, yt_ref):
    deg = jnp.sum(degp_ref[...], axis=0, keepdims=True) + 1.0
    dinv = lax.rsqrt(deg)
    dinv_ref[...] = dinv
    yt = lax.dot_general(w_ref[...], x_ref[...], (((0,), (1,)), ((), ())),
                         preferred_element_type=jnp.float32)
    yt_ref[...] = yt * dinv


def _tc1(degp, xp, w):
    return pl.pallas_call(
        _tc1_body,
        grid=(NP // _R,),
        in_specs=[
            pl.BlockSpec((NW, _R), lambda i: (0, i)),
            pl.BlockSpec((_R, D), lambda i: (i, 0)),
            pl.BlockSpec((D, H), lambda i: (0, 0)),
        ],
        out_specs=[
            pl.BlockSpec((1, _R), lambda i: (0, i)),
            pl.BlockSpec((H, _R), lambda i: (0, i)),
        ],
        out_shape=[
            jax.ShapeDtypeStruct((1, NP), jnp.float32),
            jax.ShapeDtypeStruct((H, NP), jnp.float32),
        ],
    )(degp, xp, w)


def _tcmid_body(st_ref, yt_ref, dinv_ref, b_ref, w_ref, ynt_ref):
    dinv = dinv_ref[...]
    z = st_ref[...] + yt_ref[...]
    h = jnp.maximum(z * dinv + b_ref[...], 0.0)
    yn = lax.dot_general(w_ref[...], h, (((0,), (0,)), ((), ())),
                         preferred_element_type=jnp.float32)
    ynt_ref[...] = yn * dinv


def _tcmid(st, yt, dinv, b, w):
    return pl.pallas_call(
        _tcmid_body,
        grid=(NP // _R,),
        in_specs=[
            pl.BlockSpec((H, _R), lambda i: (0, i)),
            pl.BlockSpec((H, _R), lambda i: (0, i)),
            pl.BlockSpec((1, _R), lambda i: (0, i)),
            pl.BlockSpec((H, 1), lambda i: (0, 0)),
            pl.BlockSpec((H, H), lambda i: (0, 0)),
        ],
        out_specs=pl.BlockSpec((H, _R), lambda i: (0, i)),
        out_shape=jax.ShapeDtypeStruct((H, NP), jnp.float32),
    )(st, yt, dinv, b, w)


def _tcf_body(st_ref, yt_ref, dinv_ref, b_ref, batch_ref, wlin_ref, blin_ref,
              out_ref, pooled_acc, cnt_acc):
    i = pl.program_id(0)

    @pl.when(i == 0)
    def _():
        pooled_acc[...] = jnp.zeros_like(pooled_acc)
        cnt_acc[...] = jnp.zeros_like(cnt_acc)

    z = (st_ref[...] + yt_ref[...]) * dinv_ref[...] + b_ref[...]
    bb = batch_ref[...]
    gi = lax.broadcasted_iota(jnp.int32, (1, G), 1)
    mt = (bb == gi).astype(jnp.float32)
    pooled_acc[...] += jnp.dot(z, mt, preferred_element_type=jnp.float32)
    cnt_acc[...] += jnp.sum(mt, axis=0, keepdims=True)

    @pl.when(i == pl.num_programs(0) - 1)
    def _():
        pooled = pooled_acc[...] / jnp.maximum(cnt_acc[...], 1.0)
        out_ref[...] = lax.dot_general(
            pooled, wlin_ref[...], (((0,), (0,)), ((), ())),
            preferred_element_type=jnp.float32) + blin_ref[...]


def _tcf(st, yt, dinv, b, batchp, wlin, blin):
    return pl.pallas_call(
        _tcf_body,
        grid=(NP // _R,),
        in_specs=[
            pl.BlockSpec((H, _R), lambda i: (0, i)),
            pl.BlockSpec((H, _R), lambda i: (0, i)),
            pl.BlockSpec((1, _R), lambda i: (0, i)),
            pl.BlockSpec((H, 1), lambda i: (0, 0)),
            pl.BlockSpec((_R, 1), lambda i: (i, 0)),
            pl.BlockSpec((H, C), lambda i: (0, 0)),
            pl.BlockSpec((1, C), lambda i: (0, 0)),
        ],
        out_specs=pl.BlockSpec((G, C), lambda i: (0, 0)),
        out_shape=jax.ShapeDtypeStruct((G, C), jnp.float32),
        scratch_shapes=[
            pltpu.VMEM((H, G), jnp.float32),
            pltpu.VMEM((1, G), jnp.float32),
        ],
    )(st, yt, dinv, b, batchp, wlin, blin)


def kernel(x, edge_index, batch, W1, b1, W2, b2, W3, b3, Wlin, blin):
    dstd_r = edge_index[1].reshape(NW, EDEG)
    eidx_r = jnp.pad(edge_index, ((0, 0), (0, EPADA - E)),
                     constant_values=N).reshape(2, NWINE, EW).transpose(1, 0, 2)
    xp = jnp.pad(x, ((0, NP - N), (0, 0)))
    batchp = jnp.pad(batch, (0, NP - N), constant_values=G).reshape(NP, 1)
    zeros1d = jnp.zeros((NP,), jnp.float32)
    zerosfs = jnp.zeros((FS, NP), jnp.float32)

    degp = _deg(dstd_r, zeros1d)
    dinv, yt = _tc1(degp, xp, W1)
    st = _agg(yt.reshape(NW, FS, NP), eidx_r, zerosfs).reshape(H, NP)
    yt = _tcmid(st, yt, dinv, b1.reshape(H, 1), W2)
    st = _agg(yt.reshape(NW, FS, NP), eidx_r, zerosfs).reshape(H, NP)
    yt = _tcmid(st, yt, dinv, b2.reshape(H, 1), W3)
    st = _agg(yt.reshape(NW, FS, NP), eidx_r, zerosfs).reshape(H, NP)
    return _tcf(st, yt, dinv, b3.reshape(H, 1), batchp,
                Wlin, blin.reshape(1, C))


# speedup vs baseline: 2.4556x; 1.0106x over previous
import jax
import jax.numpy as jnp
from jax import lax
from jax.experimental import pallas as pl
from jax.experimental.pallas import tpu as pltpu
from jax.experimental.pallas import tpu_sc as plsc

N = 10000
E = 320000
D = 128
H = 128
G = 64
C = 2

NC = 2
NS = 16
NW = NC * NS
NP = 10240
FS = H // NW
EDEG = E // NW
EW = 4096
NWINE = 79
EPADA = NWINE * EW

_mesh = plsc.VectorSubcoreMesh(
    core_axis_name="c", subcore_axis_name="s", num_cores=NC, num_subcores=NS)


def _deg_body(dst_hbm, zeros_hbm, out_hbm, dst_v, deg_v):
    c = lax.axis_index("c")
    s = lax.axis_index("s")
    wid = s * NC + c
    pltpu.sync_copy(dst_hbm.at[wid], dst_v)
    pltpu.sync_copy(zeros_hbm, deg_v)
    ones = jnp.ones((16,), jnp.float32)

    def body(k, carry):
        idx = dst_v[pl.ds(k * 16, 16)]
        plsc.addupdate_scatter(deg_v, [idx], ones)
        return carry

    lax.fori_loop(0, EDEG // 16, body, 0)
    pltpu.sync_copy(deg_v, out_hbm.at[wid])


_deg = pl.kernel(
    _deg_body,
    out_type=jax.ShapeDtypeStruct((NW, NP), jnp.float32),
    mesh=_mesh,
    compiler_params=pltpu.CompilerParams(needs_layout_passes=False),
    scratch_types=[
        pltpu.VMEM((EDEG,), jnp.int32),
        pltpu.VMEM((NP,), jnp.float32),
    ],
)


def _agg_body(yt_hbm, eidx_hbm, out_hbm,
              ytab, acc, ib0, ib1, sem0, sem1):
    c = lax.axis_index("c")
    s = lax.axis_index("s")
    wid = s * NC + c
    pltpu.async_copy(eidx_hbm.at[0], ib0, sem0)
    pltpu.sync_copy(yt_hbm.at[wid], ytab)

    z16 = jnp.zeros((16,), jnp.float32)

    @plsc.parallel_loop(0, FS * NP, step=16, unroll=8)
    def zero(i):
        acc[pl.ds(i, 16)] = z16

    def process(ib):
        @plsc.parallel_loop(0, EW, step=16, unroll=4)
        def grp(i):
            src16 = ib[0, pl.ds(i, 16)]
            dst16 = ib[1, pl.ds(i, 16)]
            for f in range(FS):
                v = plsc.load_gather(ytab, [src16 + (f * NP)])
                plsc.addupdate_scatter(acc, [dst16 + (f * NP)], v)

    def dbl(k, carry):
        w0 = 2 * k
        pltpu.make_async_copy(eidx_hbm.at[w0], ib0, sem0).wait()
        pltpu.async_copy(eidx_hbm.at[w0 + 1], ib1, sem1)
        process(ib0)
        pltpu.make_async_copy(eidx_hbm.at[w0 + 1], ib1, sem1).wait()
        pltpu.async_copy(eidx_hbm.at[w0 + 2], ib0, sem0)
        process(ib1)
        return carry

    lax.fori_loop(0, NWINE // 2, dbl, 0)
    pltpu.make_async_copy(eidx_hbm.at[NWINE - 1], ib0, sem0).wait()
    process(ib0)
    pltpu.sync_copy(acc, out_hbm.at[wid])


_agg = pl.kernel(
    _agg_body,
    out_type=jax.ShapeDtypeStruct((NW, FS * NP), jnp.float32),
    mesh=_mesh,
    compiler_params=pltpu.CompilerParams(needs_layout_passes=False),
    scratch_types=[
        pltpu.VMEM((FS * NP,), jnp.float32),
        pltpu.VMEM((FS * NP,), jnp.float32),
        pltpu.VMEM((2, EW), jnp.int32),
        pltpu.VMEM((2, EW), jnp.int32),
        pltpu.SemaphoreType.DMA,
        pltpu.SemaphoreType.DMA,
    ],
)


_R = 2048


def _tc1_body(degp_ref, x_ref, w_ref, dinv_ref, yt_ref):
    deg = jnp.sum(degp_ref[...], axis=0, keepdims=True) + 1.0
    dinv = lax.rsqrt(deg)
    dinv_ref[...] = dinv
    yt = lax.dot_general(w_ref[...], x_ref[...], (((0,), (1,)), ((), ())),
                         preferred_element_type=jnp.float32)
    yt_ref[...] = yt * dinv


def _tc1(degp, xp, w):
    return pl.pallas_call(
        _tc1_body,
        grid=(NP // _R,),
        in_specs=[
            pl.BlockSpec((NW, _R), lambda i: (0, i)),
            pl.BlockSpec((_R, D), lambda i: (i, 0)),
            pl.BlockSpec((D, H), lambda i: (0, 0)),
        ],
        out_specs=[
            pl.BlockSpec((1, _R), lambda i: (0, i)),
            pl.BlockSpec((H, _R), lambda i: (0, i)),
        ],
        out_shape=[
            jax.ShapeDtypeStruct((1, NP), jnp.float32),
            jax.ShapeDtypeStruct((H, NP), jnp.float32),
        ],
    )(degp, xp, w)


def _tcmid_body(st_ref, yt_ref, dinv_ref, b_ref, w_ref, ynt_ref):
    dinv = dinv_ref[...]
    z = st_ref[...] + yt_ref[...]
    h = jnp.maximum(z * dinv + b_ref[...], 0.0)
    yn = lax.dot_general(w_ref[...], h, (((0,), (0,)), ((), ())),
                         preferred_element_type=jnp.float32)
    ynt_ref[...] = yn * dinv


def _tcmid(st, yt, dinv, b, w):
    return pl.pallas_call(
        _tcmid_body,
        grid=(NP // _R,),
        in_specs=[
            pl.BlockSpec((H, _R), lambda i: (0, i)),
            pl.BlockSpec((H, _R), lambda i: (0, i)),
            pl.BlockSpec((1, _R), lambda i: (0, i)),
            pl.BlockSpec((H, 1), lambda i: (0, 0)),
            pl.BlockSpec((H, H), lambda i: (0, 0)),
        ],
        out_specs=pl.BlockSpec((H, _R), lambda i: (0, i)),
        out_shape=jax.ShapeDtypeStruct((H, NP), jnp.float32),
    )(st, yt, dinv, b, w)


def _tcf_body(st_ref, yt_ref, dinv_ref, b_ref, batch_ref, wlin_ref, blin_ref,
              out_ref, pooled_acc, cnt_acc):
    i = pl.program_id(0)

    @pl.when(i == 0)
    def _():
        pooled_acc[...] = jnp.zeros_like(pooled_acc)
        cnt_acc[...] = jnp.zeros_like(cnt_acc)

    z = (st_ref[...] + yt_ref[...]) * dinv_ref[...] + b_ref[...]
    bb = batch_ref[...]
    gi = lax.broadcasted_iota(jnp.int32, (1, G), 1)
    mt = (bb == gi).astype(jnp.float32)
    pooled_acc[...] += jnp.dot(z, mt, preferred_element_type=jnp.float32)
    cnt_acc[...] += jnp.sum(mt, axis=0, keepdims=True)

    @pl.when(i == pl.num_programs(0) - 1)
    def _():
        pooled = pooled_acc[...] / jnp.maximum(cnt_acc[...], 1.0)
        out_ref[...] = lax.dot_general(
            pooled, wlin_ref[...], (((0,), (0,)), ((), ())),
            preferred_element_type=jnp.float32) + blin_ref[...]


def _tcf(st, yt, dinv, b, batchp, wlin, blin):
    return pl.pallas_call(
        _tcf_body,
        grid=(NP // _R,),
        in_specs=[
            pl.BlockSpec((H, _R), lambda i: (0, i)),
            pl.BlockSpec((H, _R), lambda i: (0, i)),
            pl.BlockSpec((1, _R), lambda i: (0, i)),
            pl.BlockSpec((H, 1), lambda i: (0, 0)),
            pl.BlockSpec((_R, 1), lambda i: (i, 0)),
            pl.BlockSpec((H, C), lambda i: (0, 0)),
            pl.BlockSpec((1, C), lambda i: (0, 0)),
        ],
        out_specs=pl.BlockSpec((G, C), lambda i: (0, 0)),
        out_shape=jax.ShapeDtypeStruct((G, C), jnp.float32),
        scratch_shapes=[
            pltpu.VMEM((H, G), jnp.float32),
            pltpu.VMEM((1, G), jnp.float32),
        ],
    )(st, yt, dinv, b, batchp, wlin, blin)


def kernel(x, edge_index, batch, W1, b1, W2, b2, W3, b3, Wlin, blin):
    dstd_r = edge_index[1].reshape(NW, EDEG)
    eidx_r = jnp.pad(edge_index, ((0, 0), (0, EPADA - E)),
                     constant_values=N).reshape(2, NWINE, EW).transpose(1, 0, 2)
    xp = jnp.pad(x, ((0, NP - N), (0, 0)))
    batchp = jnp.pad(batch, (0, NP - N), constant_values=G).reshape(NP, 1)
    zeros1d = jnp.zeros((NP,), jnp.float32)

    degp = _deg(dstd_r, zeros1d)
    dinv, yt = _tc1(degp, xp, W1)
    st = _agg(yt.reshape(NW, FS * NP), eidx_r).reshape(H, NP)
    yt = _tcmid(st, yt, dinv, b1.reshape(H, 1), W2)
    st = _agg(yt.reshape(NW, FS * NP), eidx_r).reshape(H, NP)
    yt = _tcmid(st, yt, dinv, b2.reshape(H, 1), W3)
    st = _agg(yt.reshape(NW, FS * NP), eidx_r).reshape(H, NP)
    return _tcf(st, yt, dinv, b3.reshape(H, 1), batchp,
                Wlin, blin.reshape(1, C))


# speedup vs baseline: 2.4783x; 1.0092x over previous
import jax
import jax.numpy as jnp
from jax import lax
from jax.experimental import pallas as pl
from jax.experimental.pallas import tpu as pltpu
from jax.experimental.pallas import tpu_sc as plsc

N = 10000
E = 320000
D = 128
H = 128
G = 64
C = 2

NC = 2
NS = 16
NW = NC * NS
NP = 10240
FS = H // NW
EDEG = E // NW
EW = 4096
NWINE = 79
EPADA = NWINE * EW

_mesh = plsc.VectorSubcoreMesh(
    core_axis_name="c", subcore_axis_name="s", num_cores=NC, num_subcores=NS)


def _deg_body(dst_hbm, zeros_hbm, out_hbm, dst_v, deg_v):
    c = lax.axis_index("c")
    s = lax.axis_index("s")
    wid = s * NC + c
    pltpu.sync_copy(dst_hbm.at[wid], dst_v)
    pltpu.sync_copy(zeros_hbm, deg_v)
    ones = jnp.ones((16,), jnp.float32)

    def body(k, carry):
        idx = dst_v[pl.ds(k * 16, 16)]
        plsc.addupdate_scatter(deg_v, [idx], ones)
        return carry

    lax.fori_loop(0, EDEG // 16, body, 0)
    pltpu.sync_copy(deg_v, out_hbm.at[wid])


_deg = pl.kernel(
    _deg_body,
    out_type=jax.ShapeDtypeStruct((NW, NP), jnp.float32),
    mesh=_mesh,
    compiler_params=pltpu.CompilerParams(needs_layout_passes=False),
    scratch_types=[
        pltpu.VMEM((EDEG,), jnp.int32),
        pltpu.VMEM((NP,), jnp.float32),
    ],
)


def _agg_body(yt_hbm, eidx_hbm, out_hbm,
              ytab, acc, ib0, ib1, sem0, sem1):
    c = lax.axis_index("c")
    s = lax.axis_index("s")
    wid = s * NC + c
    pltpu.async_copy(eidx_hbm.at[0], ib0, sem0)
    pltpu.sync_copy(yt_hbm.at[wid], ytab)

    z16 = jnp.zeros((16,), jnp.float32)

    @plsc.parallel_loop(0, FS * NP, step=16, unroll=8)
    def zero(i):
        acc[pl.ds(i, 16)] = z16

    def process(ib):
        @plsc.parallel_loop(0, EW, step=16, unroll=8)
        def grp(i):
            src16 = ib[0, pl.ds(i, 16)]
            dst16 = ib[1, pl.ds(i, 16)]
            for f in range(FS):
                v = plsc.load_gather(ytab, [src16 + (f * NP)])
                plsc.addupdate_scatter(acc, [dst16 + (f * NP)], v)

    def dbl(k, carry):
        w0 = 2 * k
        pltpu.make_async_copy(eidx_hbm.at[w0], ib0, sem0).wait()
        pltpu.async_copy(eidx_hbm.at[w0 + 1], ib1, sem1)
        process(ib0)
        pltpu.make_async_copy(eidx_hbm.at[w0 + 1], ib1, sem1).wait()
        pltpu.async_copy(eidx_hbm.at[w0 + 2], ib0, sem0)
        process(ib1)
        return carry

    lax.fori_loop(0, NWINE // 2, dbl, 0)
    pltpu.make_async_copy(eidx_hbm.at[NWINE - 1], ib0, sem0).wait()
    process(ib0)
    pltpu.sync_copy(acc, out_hbm.at[wid])


_agg = pl.kernel(
    _agg_body,
    out_type=jax.ShapeDtypeStruct((NW, FS * NP), jnp.float32),
    mesh=_mesh,
    compiler_params=pltpu.CompilerParams(needs_layout_passes=False),
    scratch_types=[
        pltpu.VMEM((FS * NP,), jnp.float32),
        pltpu.VMEM((FS * NP,), jnp.float32),
        pltpu.VMEM((2, EW), jnp.int32),
        pltpu.VMEM((2, EW), jnp.int32),
        pltpu.SemaphoreType.DMA,
        pltpu.SemaphoreType.DMA,
    ],
)


_R = 2048


def _tc1_body(degp_ref, x_ref, w_ref, dinv_ref, yt_ref):
    deg = jnp.sum(degp_ref[...], axis=0, keepdims=True) + 1.0
    dinv = lax.rsqrt(deg)
    dinv_ref[...] = dinv
    yt = lax.dot_general(w_ref[...], x_ref[...], (((0,), (1,)), ((), ())),
                         preferred_element_type=jnp.float32)
    yt_ref[...] = yt * dinv


def _tc1(degp, xp, w):
    return pl.pallas_call(
        _tc1_body,
        grid=(NP // _R,),
        in_specs=[
            pl.BlockSpec((NW, _R), lambda i: (0, i)),
            pl.BlockSpec((_R, D), lambda i: (i, 0)),
            pl.BlockSpec((D, H), lambda i: (0, 0)),
        ],
        out_specs=[
            pl.BlockSpec((1, _R), lambda i: (0, i)),
            pl.BlockSpec((H, _R), lambda i: (0, i)),
        ],
        out_shape=[
            jax.ShapeDtypeStruct((1, NP), jnp.float32),
            jax.ShapeDtypeStruct((H, NP), jnp.float32),
        ],
    )(degp, xp, w)


def _tcmid_body(st_ref, yt_ref, dinv_ref, b_ref, w_ref, ynt_ref):
    dinv = dinv_ref[...]
    z = st_ref[...] + yt_ref[...]
    h = jnp.maximum(z * dinv + b_ref[...], 0.0)
    yn = lax.dot_general(w_ref[...], h, (((0,), (0,)), ((), ())),
                         preferred_element_type=jnp.float32)
    ynt_ref[...] = yn * dinv


def _tcmid(st, yt, dinv, b, w):
    return pl.pallas_call(
        _tcmid_body,
        grid=(NP // _R,),
        in_specs=[
            pl.BlockSpec((H, _R), lambda i: (0, i)),
            pl.BlockSpec((H, _R), lambda i: (0, i)),
            pl.BlockSpec((1, _R), lambda i: (0, i)),
            pl.BlockSpec((H, 1), lambda i: (0, 0)),
            pl.BlockSpec((H, H), lambda i: (0, 0)),
        ],
        out_specs=pl.BlockSpec((H, _R), lambda i: (0, i)),
        out_shape=jax.ShapeDtypeStruct((H, NP), jnp.float32),
    )(st, yt, dinv, b, w)


def _tcf_body(st_ref, yt_ref, dinv_ref, b_ref, batch_ref, wlin_ref, blin_ref,
              out_ref, pooled_acc, cnt_acc):
    i = pl.program_id(0)

    @pl.when(i == 0)
    def _():
        pooled_acc[...] = jnp.zeros_like(pooled_acc)
        cnt_acc[...] = jnp.zeros_like(cnt_acc)

    z = (st_ref[...] + yt_ref[...]) * dinv_ref[...] + b_ref[...]
    bb = batch_ref[...]
    gi = lax.broadcasted_iota(jnp.int32, (1, G), 1)
    mt = (bb == gi).astype(jnp.float32)
    pooled_acc[...] += jnp.dot(z, mt, preferred_element_type=jnp.float32)
    cnt_acc[...] += jnp.sum(mt, axis=0, keepdims=True)

    @pl.when(i == pl.num_programs(0) - 1)
    def _():
        pooled = pooled_acc[...] / jnp.maximum(cnt_acc[...], 1.0)
        out_ref[...] = lax.dot_general(
            pooled, wlin_ref[...], (((0,), (0,)), ((), ())),
            preferred_element_type=jnp.float32) + blin_ref[...]


def _tcf(st, yt, dinv, b, batchp, wlin, blin):
    return pl.pallas_call(
        _tcf_body,
        grid=(NP // _R,),
        in_specs=[
            pl.BlockSpec((H, _R), lambda i: (0, i)),
            pl.BlockSpec((H, _R), lambda i: (0, i)),
            pl.BlockSpec((1, _R), lambda i: (0, i)),
            pl.BlockSpec((H, 1), lambda i: (0, 0)),
            pl.BlockSpec((_R, 1), lambda i: (i, 0)),
            pl.BlockSpec((H, C), lambda i: (0, 0)),
            pl.BlockSpec((1, C), lambda i: (0, 0)),
        ],
        out_specs=pl.BlockSpec((G, C), lambda i: (0, 0)),
        out_shape=jax.ShapeDtypeStruct((G, C), jnp.float32),
        scratch_shapes=[
            pltpu.VMEM((H, G), jnp.float32),
            pltpu.VMEM((1, G), jnp.float32),
        ],
    )(st, yt, dinv, b, batchp, wlin, blin)


def kernel(x, edge_index, batch, W1, b1, W2, b2, W3, b3, Wlin, blin):
    dstd_r = edge_index[1].reshape(NW, EDEG)
    eidx_r = jnp.pad(edge_index, ((0, 0), (0, EPADA - E)),
                     constant_values=N).reshape(2, NWINE, EW).transpose(1, 0, 2)
    xp = jnp.pad(x, ((0, NP - N), (0, 0)))
    batchp = jnp.pad(batch, (0, NP - N), constant_values=G).reshape(NP, 1)
    zeros1d = jnp.zeros((NP,), jnp.float32)

    degp = _deg(dstd_r, zeros1d)
    dinv, yt = _tc1(degp, xp, W1)
    st = _agg(yt.reshape(NW, FS * NP), eidx_r).reshape(H, NP)
    yt = _tcmid(st, yt, dinv, b1.reshape(H, 1), W2)
    st = _agg(yt.reshape(NW, FS * NP), eidx_r).reshape(H, NP)
    yt = _tcmid(st, yt, dinv, b2.reshape(H, 1), W3)
    st = _agg(yt.reshape(NW, FS * NP), eidx_r).reshape(H, NP)
    return _tcf(st, yt, dinv, b3.reshape(H, 1), batchp,
                Wlin, blin.reshape(1, C))


# speedup vs baseline: 2.6364x; 1.0638x over previous
import jax
import jax.numpy as jnp
from jax import lax
from jax.experimental import pallas as pl
from jax.experimental.pallas import tpu as pltpu
from jax.experimental.pallas import tpu_sc as plsc

N = 10000
E = 320000
D = 128
H = 128
G = 64
C = 2

NC = 2
NS = 16
NW = NC * NS
NP = 10240
FS = H // NW
EDEG = E // NW
EW = 4096
NWINE = 79
EPADA = NWINE * EW

_mesh = plsc.VectorSubcoreMesh(
    core_axis_name="c", subcore_axis_name="s", num_cores=NC, num_subcores=NS)


def _deg_body(dst_hbm, zeros_hbm, out_hbm, dst_v, deg_v):
    c = lax.axis_index("c")
    s = lax.axis_index("s")
    wid = s * NC + c
    pltpu.sync_copy(dst_hbm.at[wid], dst_v)
    pltpu.sync_copy(zeros_hbm, deg_v)
    ones = jnp.ones((16,), jnp.float32)

    def body(k, carry):
        idx = dst_v[pl.ds(k * 16, 16)]
        plsc.addupdate_scatter(deg_v, [idx], ones)
        return carry

    lax.fori_loop(0, EDEG // 16, body, 0)
    pltpu.sync_copy(deg_v, out_hbm.at[wid])


_deg = pl.kernel(
    _deg_body,
    out_type=jax.ShapeDtypeStruct((NW, NP), jnp.float32),
    mesh=_mesh,
    compiler_params=pltpu.CompilerParams(needs_layout_passes=False),
    scratch_types=[
        pltpu.VMEM((EDEG,), jnp.int32),
        pltpu.VMEM((NP,), jnp.float32),
    ],
)


def _agg_body(yp_hbm, eidx_hbm, out_hbm,
              yptab, acc, ib0, ib1, sem0, sem1):
    c = lax.axis_index("c")
    s = lax.axis_index("s")
    wid = s * NC + c
    pltpu.async_copy(eidx_hbm.at[0, 0], ib0, sem0)
    pltpu.sync_copy(yp_hbm.at[wid, 0], yptab)

    z16 = jnp.zeros((16,), jnp.float32)

    @plsc.parallel_loop(0, FS * NP, step=16, unroll=8)
    def zero(i):
        acc[pl.ds(i, 16)] = z16

    def process(ib):
        @plsc.parallel_loop(0, EW, step=16, unroll=8)
        def grp(i):
            ew = ib[pl.ds(i, 16)]
            src16 = ew & 0xFFFF
            dst16 = lax.shift_right_logical(ew, 16)
            for f2 in range(2):
                w = plsc.load_gather(yptab, [src16 + (f2 * NP)])
                vlo = plsc.bitcast(lax.shift_left(w, 16), jnp.float32)
                vhi = plsc.bitcast(w & jnp.int32(-65536), jnp.float32)
                plsc.addupdate_scatter(acc, [dst16 + (f2 * NP)], vlo)
                plsc.addupdate_scatter(
                    acc, [dst16 + ((2 + f2) * NP)], vhi)

    def dbl(k, carry):
        w0 = 2 * k
        pltpu.make_async_copy(eidx_hbm.at[w0, 0], ib0, sem0).wait()
        pltpu.async_copy(eidx_hbm.at[w0 + 1, 0], ib1, sem1)
        process(ib0)
        pltpu.make_async_copy(eidx_hbm.at[w0 + 1, 0], ib1, sem1).wait()
        pltpu.async_copy(eidx_hbm.at[w0 + 2, 0], ib0, sem0)
        process(ib1)
        return carry

    lax.fori_loop(0, NWINE // 2, dbl, 0)
    pltpu.make_async_copy(eidx_hbm.at[NWINE - 1, 0], ib0, sem0).wait()
    process(ib0)
    pltpu.sync_copy(acc, out_hbm.at[wid])


_agg = pl.kernel(
    _agg_body,
    out_type=jax.ShapeDtypeStruct((NW, FS * NP), jnp.float32),
    mesh=_mesh,
    compiler_params=pltpu.CompilerParams(needs_layout_passes=False),
    scratch_types=[
        pltpu.VMEM((2 * NP,), jnp.int32),
        pltpu.VMEM((FS * NP,), jnp.float32),
        pltpu.VMEM((EW,), jnp.int32),
        pltpu.VMEM((EW,), jnp.int32),
        pltpu.SemaphoreType.DMA,
        pltpu.SemaphoreType.DMA,
    ],
)


_R = 2048


def _pack_pairs(yt):
    ub = lax.bitcast_convert_type(
        yt[:H // 2].astype(jnp.bfloat16), jnp.uint16).astype(jnp.uint32)
    ut = lax.bitcast_convert_type(
        yt[H // 2:].astype(jnp.bfloat16), jnp.uint16).astype(jnp.uint32)
    return lax.bitcast_convert_type((ut << 16) | ub, jnp.int32)


def _tc1_body(degp_ref, x_ref, w_ref, dinv_ref, yt_ref, yp_ref):
    deg = jnp.sum(degp_ref[...], axis=0, keepdims=True) + 1.0
    dinv = lax.rsqrt(deg)
    dinv_ref[...] = dinv
    yt = lax.dot_general(w_ref[...], x_ref[...], (((0,), (1,)), ((), ())),
                         preferred_element_type=jnp.float32) * dinv
    yt_ref[...] = yt
    yp_ref[...] = _pack_pairs(yt)


def _tc1(degp, xp, w):
    return pl.pallas_call(
        _tc1_body,
        grid=(NP // _R,),
        in_specs=[
            pl.BlockSpec((NW, _R), lambda i: (0, i)),
            pl.BlockSpec((_R, D), lambda i: (i, 0)),
            pl.BlockSpec((D, H), lambda i: (0, 0)),
        ],
        out_specs=[
            pl.BlockSpec((1, _R), lambda i: (0, i)),
            pl.BlockSpec((H, _R), lambda i: (0, i)),
            pl.BlockSpec((H // 2, _R), lambda i: (0, i)),
        ],
        out_shape=[
            jax.ShapeDtypeStruct((1, NP), jnp.float32),
            jax.ShapeDtypeStruct((H, NP), jnp.float32),
            jax.ShapeDtypeStruct((H // 2, NP), jnp.int32),
        ],
    )(degp, xp, w)


def _tcmid_body(st_ref, yt_ref, dinv_ref, b_ref, w_ref, ynt_ref, ynp_ref):
    dinv = dinv_ref[...]
    z = st_ref[...] + yt_ref[...]
    h = jnp.maximum(z * dinv + b_ref[...], 0.0)
    yn = lax.dot_general(w_ref[...], h, (((0,), (0,)), ((), ())),
                         preferred_element_type=jnp.float32) * dinv
    ynt_ref[...] = yn
    ynp_ref[...] = _pack_pairs(yn)


def _tcmid(st, yt, dinv, b, w):
    return pl.pallas_call(
        _tcmid_body,
        grid=(NP // _R,),
        in_specs=[
            pl.BlockSpec((H, _R), lambda i: (0, i)),
            pl.BlockSpec((H, _R), lambda i: (0, i)),
            pl.BlockSpec((1, _R), lambda i: (0, i)),
            pl.BlockSpec((H, 1), lambda i: (0, 0)),
            pl.BlockSpec((H, H), lambda i: (0, 0)),
        ],
        out_specs=[
            pl.BlockSpec((H, _R), lambda i: (0, i)),
            pl.BlockSpec((H // 2, _R), lambda i: (0, i)),
        ],
        out_shape=[
            jax.ShapeDtypeStruct((H, NP), jnp.float32),
            jax.ShapeDtypeStruct((H // 2, NP), jnp.int32),
        ],
    )(st, yt, dinv, b, w)


def _tcf_body(st_ref, yt_ref, dinv_ref, b_ref, batch_ref, wlin_ref, blin_ref,
              out_ref, pooled_acc, cnt_acc):
    i = pl.program_id(0)

    @pl.when(i == 0)
    def _():
        pooled_acc[...] = jnp.zeros_like(pooled_acc)
        cnt_acc[...] = jnp.zeros_like(cnt_acc)

    z = (st_ref[...] + yt_ref[...]) * dinv_ref[...] + b_ref[...]
    bb = batch_ref[...]
    gi = lax.broadcasted_iota(jnp.int32, (1, G), 1)
    mt = (bb == gi).astype(jnp.float32)
    pooled_acc[...] += jnp.dot(z, mt, preferred_element_type=jnp.float32)
    cnt_acc[...] += jnp.sum(mt, axis=0, keepdims=True)

    @pl.when(i == pl.num_programs(0) - 1)
    def _():
        pooled = pooled_acc[...] / jnp.maximum(cnt_acc[...], 1.0)
        out_ref[...] = lax.dot_general(
            pooled, wlin_ref[...], (((0,), (0,)), ((), ())),
            preferred_element_type=jnp.float32) + blin_ref[...]


def _tcf(st, yt, dinv, b, batchp, wlin, blin):
    return pl.pallas_call(
        _tcf_body,
        grid=(NP // _R,),
        in_specs=[
            pl.BlockSpec((H, _R), lambda i: (0, i)),
            pl.BlockSpec((H, _R), lambda i: (0, i)),
            pl.BlockSpec((1, _R), lambda i: (0, i)),
            pl.BlockSpec((H, 1), lambda i: (0, 0)),
            pl.BlockSpec((_R, 1), lambda i: (i, 0)),
            pl.BlockSpec((H, C), lambda i: (0, 0)),
            pl.BlockSpec((1, C), lambda i: (0, 0)),
        ],
        out_specs=pl.BlockSpec((G, C), lambda i: (0, 0)),
        out_shape=jax.ShapeDtypeStruct((G, C), jnp.float32),
        scratch_shapes=[
            pltpu.VMEM((H, G), jnp.float32),
            pltpu.VMEM((1, G), jnp.float32),
        ],
    )(st, yt, dinv, b, batchp, wlin, blin)


def _unperm(s):
    return s.reshape(NW, 2, 2, NP).transpose(1, 0, 2, 3).reshape(H, NP)


def kernel(x, edge_index, batch, W1, b1, W2, b2, W3, b3, Wlin, blin):
    dstd_r = edge_index[1].reshape(NW, EDEG)
    epk = (edge_index[1] << 16) | edge_index[0]
    eidx_r = jnp.pad(epk, (0, EPADA - E),
                     constant_values=(N << 16) | N).reshape(NWINE, 1, EW)
    xp = jnp.pad(x, ((0, NP - N), (0, 0)))
    batchp = jnp.pad(batch, (0, NP - N), constant_values=G).reshape(NP, 1)
    zeros1d = jnp.zeros((NP,), jnp.float32)

    degp = _deg(dstd_r, zeros1d)
    dinv, yt, yp = _tc1(degp, xp, W1)
    st = _unperm(_agg(yp.reshape(NW, 1, 2 * NP), eidx_r))
    yt, yp = _tcmid(st, yt, dinv, b1.reshape(H, 1), W2)
    st = _unperm(_agg(yp.reshape(NW, 1, 2 * NP), eidx_r))
    yt, yp = _tcmid(st, yt, dinv, b2.reshape(H, 1), W3)
    st = _unperm(_agg(yp.reshape(NW, 1, 2 * NP), eidx_r))
    return _tcf(st, yt, dinv, b3.reshape(H, 1), batchp,
                Wlin, blin.reshape(1, C))


# speedup vs baseline: 2.8863x; 1.0948x over previous
import jax
import jax.numpy as jnp
import numpy as np
from jax import lax
from jax.experimental import pallas as pl
from jax.experimental.pallas import tpu as pltpu
from jax.experimental.pallas import tpu_sc as plsc

N = 10000
E = 320000
D = 128
H = 128
G = 64
C = 2

NC = 2
NS = 16
NW = NC * NS
NP = 10240
FS = H // NW
EDEG = E // NW
EW = 4096
NWINE = 79
EPADA = NWINE * EW

_PI = np.array([2 * (q // 4) + (q % 2) + 64 * ((q % 4) // 2) for q in range(H)])

_mesh = plsc.VectorSubcoreMesh(
    core_axis_name="c", subcore_axis_name="s", num_cores=NC, num_subcores=NS)


def _deg_body(dst_hbm, zeros_hbm, out_hbm, dst_v, deg_v):
    c = lax.axis_index("c")
    s = lax.axis_index("s")
    wid = s * NC + c
    pltpu.sync_copy(dst_hbm.at[wid], dst_v)
    pltpu.sync_copy(zeros_hbm, deg_v)
    ones = jnp.ones((16,), jnp.float32)

    def body(k, carry):
        idx = dst_v[pl.ds(k * 16, 16)]
        plsc.addupdate_scatter(deg_v, [idx], ones)
        return carry

    lax.fori_loop(0, EDEG // 16, body, 0)
    pltpu.sync_copy(deg_v, out_hbm.at[wid])


_deg = pl.kernel(
    _deg_body,
    out_type=jax.ShapeDtypeStruct((NW, NP), jnp.float32),
    mesh=_mesh,
    compiler_params=pltpu.CompilerParams(needs_layout_passes=False),
    scratch_types=[
        pltpu.VMEM((EDEG,), jnp.int32),
        pltpu.VMEM((NP,), jnp.float32),
    ],
)


def _agg_body(yp_hbm, eidx_hbm, out_hbm,
              yptab, acc, ib0, ib1, sem0, sem1):
    c = lax.axis_index("c")
    s = lax.axis_index("s")
    wid = s * NC + c
    pltpu.async_copy(eidx_hbm.at[0, 0], ib0, sem0)
    pltpu.sync_copy(yp_hbm.at[wid, 0], yptab)

    z16 = jnp.zeros((16,), jnp.float32)

    @plsc.parallel_loop(0, FS * NP, step=16, unroll=8)
    def zero(i):
        acc[pl.ds(i, 16)] = z16

    def process(ib):
        @plsc.parallel_loop(0, EW, step=16, unroll=8)
        def grp(i):
            ew = ib[pl.ds(i, 16)]
            src16 = ew & 0xFFFF
            dst16 = lax.shift_right_logical(ew, 16)
            for f2 in range(2):
                w = plsc.load_gather(yptab, [src16 + (f2 * NP)])
                vlo = plsc.bitcast(lax.shift_left(w, 16), jnp.float32)
                vhi = plsc.bitcast(w & jnp.int32(-65536), jnp.float32)
                plsc.addupdate_scatter(acc, [dst16 + (f2 * NP)], vlo)
                plsc.addupdate_scatter(
                    acc, [dst16 + ((2 + f2) * NP)], vhi)

    def dbl(k, carry):
        w0 = 2 * k
        pltpu.make_async_copy(eidx_hbm.at[w0, 0], ib0, sem0).wait()
        pltpu.async_copy(eidx_hbm.at[w0 + 1, 0], ib1, sem1)
        process(ib0)
        pltpu.make_async_copy(eidx_hbm.at[w0 + 1, 0], ib1, sem1).wait()
        pltpu.async_copy(eidx_hbm.at[w0 + 2, 0], ib0, sem0)
        process(ib1)
        return carry

    lax.fori_loop(0, NWINE // 2, dbl, 0)
    pltpu.make_async_copy(eidx_hbm.at[NWINE - 1, 0], ib0, sem0).wait()
    process(ib0)
    pltpu.sync_copy(acc, out_hbm.at[wid])


_agg = pl.kernel(
    _agg_body,
    out_type=jax.ShapeDtypeStruct((NW, FS * NP), jnp.float32),
    mesh=_mesh,
    compiler_params=pltpu.CompilerParams(needs_layout_passes=False),
    scratch_types=[
        pltpu.VMEM((2 * NP,), jnp.int32),
        pltpu.VMEM((FS * NP,), jnp.float32),
        pltpu.VMEM((EW,), jnp.int32),
        pltpu.VMEM((EW,), jnp.int32),
        pltpu.SemaphoreType.DMA,
        pltpu.SemaphoreType.DMA,
    ],
)


_R = 2048


def _pack_pairs(yt):
    ub = lax.bitcast_convert_type(
        yt[:H // 2].astype(jnp.bfloat16), jnp.uint16).astype(jnp.uint32)
    ut = lax.bitcast_convert_type(
        yt[H // 2:].astype(jnp.bfloat16), jnp.uint16).astype(jnp.uint32)
    return lax.bitcast_convert_type((ut << 16) | ub, jnp.int32)


def _tc1_body(degp_ref, x_ref, wq_ref, wn_ref, dinv_ref, yt_ref, yp_ref):
    deg = jnp.sum(degp_ref[...], axis=0, keepdims=True) + 1.0
    dinv = lax.rsqrt(deg)
    dinv_ref[...] = dinv
    ytq = lax.dot_general(wq_ref[...], x_ref[...], (((0,), (1,)), ((), ())),
                          preferred_element_type=jnp.float32) * dinv
    yt_ref[...] = ytq
    ytn = lax.dot_general(wn_ref[...], x_ref[...], (((0,), (1,)), ((), ())),
                          preferred_element_type=jnp.float32) * dinv
    yp_ref[...] = _pack_pairs(ytn)


def _tc1(degp, xp, wq, wn):
    return pl.pallas_call(
        _tc1_body,
        grid=(NP // _R,),
        in_specs=[
            pl.BlockSpec((NW, _R), lambda i: (0, i)),
            pl.BlockSpec((_R, D), lambda i: (i, 0)),
            pl.BlockSpec((D, H), lambda i: (0, 0)),
            pl.BlockSpec((D, H), lambda i: (0, 0)),
        ],
        out_specs=[
            pl.BlockSpec((1, _R), lambda i: (0, i)),
            pl.BlockSpec((H, _R), lambda i: (0, i)),
            pl.BlockSpec((H // 2, _R), lambda i: (0, i)),
        ],
        out_shape=[
            jax.ShapeDtypeStruct((1, NP), jnp.float32),
            jax.ShapeDtypeStruct((H, NP), jnp.float32),
            jax.ShapeDtypeStruct((H // 2, NP), jnp.int32),
        ],
    )(degp, xp, wq, wn)


def _tcmid_body(st_ref, yt_ref, dinv_ref, b_ref, wqq_ref, wqn_ref,
                ynt_ref, ynp_ref):
    dinv = dinv_ref[...]
    z = st_ref[...] + yt_ref[...]
    h = jnp.maximum(z * dinv + b_ref[...], 0.0)
    ynq = lax.dot_general(wqq_ref[...], h, (((0,), (0,)), ((), ())),
                          preferred_element_type=jnp.float32) * dinv
    ynt_ref[...] = ynq
    ynn = lax.dot_general(wqn_ref[...], h, (((0,), (0,)), ((), ())),
                          preferred_element_type=jnp.float32) * dinv
    ynp_ref[...] = _pack_pairs(ynn)


def _tcmid(st, yt, dinv, b, wqq, wqn):
    return pl.pallas_call(
        _tcmid_body,
        grid=(NP // _R,),
        in_specs=[
            pl.BlockSpec((H, _R), lambda i: (0, i)),
            pl.BlockSpec((H, _R), lambda i: (0, i)),
            pl.BlockSpec((1, _R), lambda i: (0, i)),
            pl.BlockSpec((H, 1), lambda i: (0, 0)),
            pl.BlockSpec((H, H), lambda i: (0, 0)),
            pl.BlockSpec((H, H), lambda i: (0, 0)),
        ],
        out_specs=[
            pl.BlockSpec((H, _R), lambda i: (0, i)),
            pl.BlockSpec((H // 2, _R), lambda i: (0, i)),
        ],
        out_shape=[
            jax.ShapeDtypeStruct((H, NP), jnp.float32),
            jax.ShapeDtypeStruct((H // 2, NP), jnp.int32),
        ],
    )(st, yt, dinv, b, wqq, wqn)


def _tcf_body(st_ref, yt_ref, dinv_ref, b_ref, batch_ref, wlin_ref, blin_ref,
              out_ref, pooled_acc, cnt_acc):
    i = pl.program_id(0)

    @pl.when(i == 0)
    def _():
        pooled_acc[...] = jnp.zeros_like(pooled_acc)
        cnt_acc[...] = jnp.zeros_like(cnt_acc)

    z = (st_ref[...] + yt_ref[...]) * dinv_ref[...] + b_ref[...]
    bb = batch_ref[...]
    gi = lax.broadcasted_iota(jnp.int32, (1, G), 1)
    mt = (bb == gi).astype(jnp.float32)
    pooled_acc[...] += jnp.dot(z, mt, preferred_element_type=jnp.float32)
    cnt_acc[...] += jnp.sum(mt, axis=0, keepdims=True)

    @pl.when(i == pl.num_programs(0) - 1)
    def _():
        pooled = pooled_acc[...] / jnp.maximum(cnt_acc[...], 1.0)
        out_ref[...] = lax.dot_general(
            pooled, wlin_ref[...], (((0,), (0,)), ((), ())),
            preferred_element_type=jnp.float32) + blin_ref[...]


def _tcf(st, yt, dinv, b, batchp, wlin, blin):
    return pl.pallas_call(
        _tcf_body,
        grid=(NP // _R,),
        in_specs=[
            pl.BlockSpec((H, _R), lambda i: (0, i)),
            pl.BlockSpec((H, _R), lambda i: (0, i)),
            pl.BlockSpec((1, _R), lambda i: (0, i)),
            pl.BlockSpec((H, 1), lambda i: (0, 0)),
            pl.BlockSpec((_R, 1), lambda i: (i, 0)),
            pl.BlockSpec((H, C), lambda i: (0, 0)),
            pl.BlockSpec((1, C), lambda i: (0, 0)),
        ],
        out_specs=pl.BlockSpec((G, C), lambda i: (0, 0)),
        out_shape=jax.ShapeDtypeStruct((G, C), jnp.float32),
        scratch_shapes=[
            pltpu.VMEM((H, G), jnp.float32),
            pltpu.VMEM((1, G), jnp.float32),
        ],
    )(st, yt, dinv, b, batchp, wlin, blin)


def kernel(x, edge_index, batch, W1, b1, W2, b2, W3, b3, Wlin, blin):
    dstd_r = edge_index[1].reshape(NW, EDEG)
    epk = (edge_index[1] << 16) | edge_index[0]
    eidx_r = jnp.pad(epk, (0, EPADA - E),
                     constant_values=(N << 16) | N).reshape(NWINE, 1, EW)
    xp = jnp.pad(x, ((0, NP - N), (0, 0)))
    batchp = jnp.pad(batch, (0, NP - N), constant_values=G).reshape(NP, 1)
    zeros1d = jnp.zeros((NP,), jnp.float32)

    degp = _deg(dstd_r, zeros1d)
    dinv, yt, yp = _tc1(degp, xp, W1[:, _PI], W1)
    st = _agg(yp.reshape(NW, 1, 2 * NP), eidx_r).reshape(H, NP)
    yt, yp = _tcmid(st, yt, dinv, b1[_PI].reshape(H, 1),
                    W2[_PI][:, _PI], W2[_PI])
    st = _agg(yp.reshape(NW, 1, 2 * NP), eidx_r).reshape(H, NP)
    yt, yp = _tcmid(st, yt, dinv, b2[_PI].reshape(H, 1),
                    W3[_PI][:, _PI], W3[_PI])
    st = _agg(yp.reshape(NW, 1, 2 * NP), eidx_r).reshape(H, NP)
    return _tcf(st, yt, dinv, b3[_PI].reshape(H, 1), batchp,
                Wlin[_PI], blin.reshape(1, C))
